# Initial kernel scaffold; baseline (speedup 1.0000x reference)
import jax
import jax.numpy as jnp
from jax.experimental import pallas as pl


def kernel(v_size, c_size, v_edge_index, c_edge_index, p_edge_index, n_edge_index, v_emb, c_emb, p_v2c_W, p_v2c_b, n_v2c_W, n_v2c_b, p_c2v_W, p_c2v_b, n_c2v_W, n_c2v_b, c_upd_W, c_upd_b, v_upd_W, v_upd_b):
    raise NotImplementedError("write your pallas kernel here")


# speedup vs baseline: 2.6321x; 2.6321x over previous
import functools

import jax
import jax.numpy as jnp
from jax import lax
from jax.experimental import pallas as pl
from jax.experimental.pallas import tpu as pltpu
from jax.experimental.pallas import tpu_sc as plsc

DIM = 128
SL = 16
NSL = DIM // SL
CHUNK = 128
NTILE = 16
LANES = 16


def _pad_nodes(n):
    return ((n + 2047) // 2048) * 2048


def _index_and_degree(v_ei, c_ei, p_ei, n_ei, vn, cn):
    ep = p_ei.shape[0]
    nch = ep // CHUNK
    vpad, cpad = _pad_nodes(vn), _pad_nodes(cn)
    vtr, ctr = vpad // NTILE, cpad // NTILE
    nloop = (nch + NTILE - 1) // NTILE

    out_type = (
        jax.ShapeDtypeStruct((nch, CHUNK), jnp.int32),
        jax.ShapeDtypeStruct((nch, CHUNK), jnp.int32),
        jax.ShapeDtypeStruct((nch, CHUNK), jnp.int32),
        jax.ShapeDtypeStruct((nch, CHUNK), jnp.int32),
        jax.ShapeDtypeStruct((vpad,), jnp.float32),
        jax.ShapeDtypeStruct((cpad,), jnp.float32),
        jax.ShapeDtypeStruct((vpad,), jnp.float32),
        jax.ShapeDtypeStruct((cpad,), jnp.float32),
    )

    @functools.partial(
        pl.kernel,
        out_type=out_type,
        mesh=plsc.VectorSubcoreMesh(core_axis_name="c", subcore_axis_name="s"),
        scratch_types=[
            pltpu.VMEM((CHUNK,), jnp.int32),
            pltpu.VMEM((CHUNK,), jnp.int32),
            pltpu.VMEM((CHUNK,), jnp.int32),
            pltpu.VMEM((CHUNK,), jnp.float32),
            pltpu.VMEM((vtr,), jnp.float32),
            pltpu.VMEM_SHARED((vpad,), jnp.float32),
            pltpu.VMEM_SHARED((cpad,), jnp.float32),
            pltpu.SemaphoreType.DMA,
        ],
    )
    def kern(v_hbm, c_hbm, p_hbm, n_hbm,
             pv_hbm, pc_hbm, nv_hbm, nc_hbm, pvd, pcd, nvd, ncd,
             eidx, srcv, srcc, ones, zbuf, vhist, chist, sem):
        core = lax.axis_index("c")
        sid = lax.axis_index("s")

        def fill_ones(j, _):
            ones[pl.ds(j * LANES, LANES)] = jnp.ones((LANES,), jnp.float32)
            return 0
        lax.fori_loop(0, CHUNK // LANES, fill_ones, 0)

        def fill_z(j, _):
            zbuf[pl.ds(j * LANES, LANES)] = jnp.zeros((LANES,), jnp.float32)
            return 0
        lax.fori_loop(0, vtr // LANES, fill_z, 0)

        pltpu.sync_copy(zbuf, vhist.at[pl.ds(sid * vtr, vtr)])

        def zero_c(j, _):
            pltpu.sync_copy(zbuf, chist.at[pl.ds(sid * ctr + j * vtr, vtr)])
            return 0
        lax.fori_loop(0, ctr // vtr, zero_c, 0)
        plsc.subcore_barrier()

        def run_set(e_hbm, sv_hbm, sc_hbm):
            def body(i, _):
                ci = sid + i * NTILE

                @pl.when(ci < nch)
                def _():
                    pltpu.sync_copy(e_hbm.at[pl.ds(ci * CHUNK, CHUNK)], eidx)
                    pltpu.async_copy(v_hbm.at[eidx], srcv, sem).wait()
                    pltpu.async_copy(c_hbm.at[eidx], srcc, sem).wait()
                    pltpu.sync_copy(srcv, sv_hbm.at[ci])
                    pltpu.sync_copy(srcc, sc_hbm.at[ci])
                    pltpu.sync_copy(ones, vhist.at[srcv], add=True)
                    pltpu.sync_copy(ones, chist.at[srcc], add=True)
                return 0
            lax.fori_loop(0, nloop, body, 0)

        @pl.when(core == 0)
        def _():
            run_set(p_hbm, pv_hbm, pc_hbm)

        @pl.when(core == 1)
        def _():
            run_set(n_hbm, nv_hbm, nc_hbm)

        plsc.subcore_barrier()

        @pl.when(core == 0)
        def _():
            pltpu.sync_copy(vhist.at[pl.ds(sid * vtr, vtr)],
                            pvd.at[pl.ds(sid * vtr, vtr)])
            pltpu.sync_copy(chist.at[pl.ds(sid * ctr, ctr)],
                            pcd.at[pl.ds(sid * ctr, ctr)])

        @pl.when(core == 1)
        def _():
            pltpu.sync_copy(vhist.at[pl.ds(sid * vtr, vtr)],
                            nvd.at[pl.ds(sid * vtr, vtr)])
            pltpu.sync_copy(chist.at[pl.ds(sid * ctr, ctr)],
                            ncd.at[pl.ds(sid * ctr, ctr)])

    return kern(v_ei, c_ei, p_ei, n_ei)


def _mlp_tables(emb, wp, bp, wn, bn, degp, degn):
    n = emb.shape[0]
    b = 1000
    grid = (n // b,)
    nsl = NSL

    def body(emb_ref, wp_ref, bp_ref, wn_ref, bn_ref, dp_ref, dn_ref, *outs):
        x = emb_ref[:]

        def mlp(w_ref, b_ref):
            h = x
            for i in range(w_ref.shape[0]):
                h = jnp.dot(h, w_ref[i], preferred_element_type=jnp.float32)
                h = h + b_ref[i]
                if i < w_ref.shape[0] - 1:
                    h = jnp.maximum(h, 0.0)
            return h

        mp = mlp(wp_ref, bp_ref) * lax.rsqrt(jnp.maximum(dp_ref[:], 1.0))
        mn = mlp(wn_ref, bn_ref) * lax.rsqrt(jnp.maximum(dn_ref[:], 1.0))
        for k in range(nsl):
            outs[k][:] = mp[:, SL * k:SL * k + SL]
            outs[nsl + k][:] = mn[:, SL * k:SL * k + SL]

    return pl.pallas_call(
        body,
        grid=grid,
        in_specs=[
            pl.BlockSpec((b, DIM), lambda i: (i, 0)),
            pl.BlockSpec((3, DIM, DIM), lambda i: (0, 0, 0)),
            pl.BlockSpec((3, 1, DIM), lambda i: (0, 0, 0)),
            pl.BlockSpec((3, DIM, DIM), lambda i: (0, 0, 0)),
            pl.BlockSpec((3, 1, DIM), lambda i: (0, 0, 0)),
            pl.BlockSpec((b, 1), lambda i: (i, 0)),
            pl.BlockSpec((b, 1), lambda i: (i, 0)),
        ],
        out_specs=[pl.BlockSpec((b, SL), lambda i: (i, 0))] * (2 * nsl),
        out_shape=[jax.ShapeDtypeStruct((n, SL), jnp.float32)] * (2 * nsl),
    )(emb, wp, bp.reshape(3, 1, DIM), wn, bn.reshape(3, 1, DIM), degp, degn)


def _transport(pv, pc, nv, nc, vp_t, vn_t, cp_t, cn_t, vn_nodes, cn_nodes):
    nch = pv.shape[0]
    vpad, cpad = _pad_nodes(vn_nodes), _pad_nodes(cn_nodes)
    vtr, ctr = vpad // NTILE, cpad // NTILE
    nloop = (nch + NTILE - 1) // NTILE

    out_type = (
        jax.ShapeDtypeStruct((NSL, cpad, SL), jnp.float32),
        jax.ShapeDtypeStruct((NSL, vpad, SL), jnp.float32),
        jax.ShapeDtypeStruct((NSL, cpad, SL), jnp.float32),
        jax.ShapeDtypeStruct((NSL, vpad, SL), jnp.float32),
    )

    @functools.partial(
        pl.kernel,
        out_type=out_type,
        mesh=plsc.VectorSubcoreMesh(core_axis_name="c", subcore_axis_name="s"),
        compiler_params=pltpu.CompilerParams(use_tc_tiling_on_sc=False),
        scratch_types=[
            pltpu.VMEM((CHUNK,), jnp.int32),
            pltpu.VMEM((CHUNK,), jnp.int32),
            pltpu.VMEM((CHUNK, SL), jnp.float32),
            pltpu.VMEM((CHUNK, SL), jnp.float32),
            pltpu.VMEM((vtr, SL), jnp.float32),
            pltpu.VMEM_SHARED((vpad, SL), jnp.float32),
            pltpu.VMEM_SHARED((cpad, SL), jnp.float32),
            pltpu.SemaphoreType.DMA,
        ],
    )
    def kern(*refs):
        pv_hbm, pc_hbm, nv_hbm, nc_hbm = refs[0:4]
        vp = refs[4:4 + NSL]
        vnt = refs[4 + NSL:4 + 2 * NSL]
        cp = refs[4 + 2 * NSL:4 + 3 * NSL]
        cnt = refs[4 + 3 * NSL:4 + 4 * NSL]
        paggc, paggv, naggc, naggv = refs[4 + 4 * NSL:8 + 4 * NSL]
        idxv, idxc, rows, rows2, zbuf, vacc, cacc, sem = refs[8 + 4 * NSL:]
        core = lax.axis_index("c")
        sid = lax.axis_index("s")

        def fill_z(j, _):
            r = j // (SL // LANES)
            zbuf[r, pl.ds((j % (SL // LANES)) * LANES, LANES)] = jnp.zeros(
                (LANES,), jnp.float32)
            return 0
        lax.fori_loop(0, vtr * SL // LANES, fill_z, 0)

        def one_pass(k, sv_hbm, sc_hbm, vtbl, ctbl, aggv, aggc):
            pltpu.sync_copy(zbuf, vacc.at[pl.ds(sid * vtr, vtr)])

            def zero_c(j, _):
                pltpu.sync_copy(zbuf, cacc.at[pl.ds(sid * ctr + j * vtr, vtr)])
                return 0
            lax.fori_loop(0, ctr // vtr, zero_c, 0)
            plsc.subcore_barrier()

            def body(i, _):
                ci = sid + i * NTILE

                @pl.when(ci < nch)
                def _():
                    pltpu.sync_copy(sv_hbm.at[ci], idxv)
                    pltpu.sync_copy(sc_hbm.at[ci], idxc)
                    pltpu.async_copy(vtbl.at[idxv], rows, sem).wait()
                    pltpu.sync_copy(rows, cacc.at[idxc], add=True)
                    pltpu.async_copy(ctbl.at[idxc], rows2, sem).wait()
                    pltpu.sync_copy(rows2, vacc.at[idxv], add=True)
                return 0
            lax.fori_loop(0, nloop, body, 0)
            plsc.subcore_barrier()

            pltpu.sync_copy(vacc.at[pl.ds(sid * vtr, vtr)],
                            aggv.at[k, pl.ds(sid * vtr, vtr)])
            pltpu.sync_copy(cacc.at[pl.ds(sid * ctr, ctr)],
                            aggc.at[k, pl.ds(sid * ctr, ctr)])
            plsc.subcore_barrier()

        @pl.when(core == 0)
        def _():
            for k in range(NSL):
                one_pass(k, pv_hbm, pc_hbm, vp[k], cp[k], paggv, paggc)

        @pl.when(core == 1)
        def _():
            for k in range(NSL):
                one_pass(k, nv_hbm, nc_hbm, vnt[k], cnt[k], naggv, naggc)

    return kern(pv, pc, nv, nc, *vp_t, *vn_t, *cp_t, *cn_t)


def _update(emb, pagg, nagg, degp, degn, w, bias):
    n = emb.shape[0]
    b = 1000

    def body(emb_ref, p_ref, n_ref, dp_ref, dn_ref, w_ref, b_ref, out_ref):
        acc = jnp.dot(emb_ref[:], w_ref[0:DIM, :],
                      preferred_element_type=jnp.float32)
        rp = lax.rsqrt(jnp.maximum(dp_ref[:], 1.0))
        rn = lax.rsqrt(jnp.maximum(dn_ref[:], 1.0))
        for k in range(NSL):
            acc += jnp.dot(p_ref[k] * rp,
                           w_ref[DIM + SL * k:DIM + SL * k + SL, :],
                           preferred_element_type=jnp.float32)
            acc += jnp.dot(n_ref[k] * rn,
                           w_ref[2 * DIM + SL * k:2 * DIM + SL * k + SL, :],
                           preferred_element_type=jnp.float32)
        out_ref[:] = acc + b_ref[:]

    return pl.pallas_call(
        body,
        grid=(n // b,),
        in_specs=[
            pl.BlockSpec((b, DIM), lambda i: (i, 0)),
            pl.BlockSpec((NSL, b, SL), lambda i: (0, i, 0)),
            pl.BlockSpec((NSL, b, SL), lambda i: (0, i, 0)),
            pl.BlockSpec((b, 1), lambda i: (i, 0)),
            pl.BlockSpec((b, 1), lambda i: (i, 0)),
            pl.BlockSpec((3 * DIM, DIM), lambda i: (0, 0)),
            pl.BlockSpec((1, DIM), lambda i: (0, 0)),
        ],
        out_specs=pl.BlockSpec((b, DIM), lambda i: (i, 0)),
        out_shape=jax.ShapeDtypeStruct((n, DIM), jnp.float32),
    )(emb, pagg, nagg, degp, degn, w, bias.reshape(1, DIM))


def kernel(v_size, c_size, v_edge_index, c_edge_index, p_edge_index,
           n_edge_index, v_emb, c_emb,
           p_v2c_W, p_v2c_b, n_v2c_W, n_v2c_b,
           p_c2v_W, p_c2v_b, n_c2v_W, n_c2v_b,
           c_upd_W, c_upd_b, v_upd_W, v_upd_b):
    vn, cn = v_emb.shape[0], c_emb.shape[0]

    pv, pc, nv, nc, pvd, pcd, nvd, ncd = _index_and_degree(
        v_edge_index, c_edge_index, p_edge_index, n_edge_index, vn, cn)

    dpv = pvd[:vn].reshape(vn, 1)
    dnv = nvd[:vn].reshape(vn, 1)
    dpc = pcd[:cn].reshape(cn, 1)
    dnc = ncd[:cn].reshape(cn, 1)

    v_tabs = _mlp_tables(v_emb, p_v2c_W, p_v2c_b, n_v2c_W, n_v2c_b, dpv, dnv)
    c_tabs = _mlp_tables(c_emb, p_c2v_W, p_c2v_b, n_c2v_W, n_c2v_b, dpc, dnc)
    vp_t, vn_t = v_tabs[:NSL], v_tabs[NSL:]
    cp_t, cn_t = c_tabs[:NSL], c_tabs[NSL:]

    paggc, paggv, naggc, naggv = _transport(
        pv, pc, nv, nc, vp_t, vn_t, cp_t, cn_t, vn, cn)

    c_new = _update(c_emb, paggc, naggc, dpc, dnc, c_upd_W, c_upd_b)
    v_new = _update(v_emb, paggv, naggv, dpv, dnv, v_upd_W, v_upd_b)

    return (jnp.stack([v_emb, v_new]), jnp.stack([c_emb, c_new]))
